# baseline (device time: 49548 ns/iter reference)
import jax
import jax.numpy as jnp
from jax import lax
from jax.experimental import pallas as pl
from jax.experimental.pallas import tpu as pltpu

N_DEV = 4
B, Sq, Skv = 2, 256, 256
HQ_LOCAL, DH = 4, 64
D_MODEL = 512


def kernel(x, Wq, K_ext, V_ext, Wo):
    my_pos = lax.axis_index("i")
    K_loc = lax.dynamic_slice_in_dim(K_ext, my_pos * HQ_LOCAL, HQ_LOCAL, axis=2)
    V_loc = lax.dynamic_slice_in_dim(V_ext, my_pos * HQ_LOCAL, HQ_LOCAL, axis=2)
    K2 = K_loc.reshape(B, Skv, HQ_LOCAL * DH)
    V2 = V_loc.reshape(B, Skv, HQ_LOCAL * DH)
    x2 = x.reshape(B * Sq, D_MODEL)

    def body(x_ref, wq_ref, k_ref, v_ref, wo_ref, out_ref,
             comm_ref, send_sems, recv_sems):
        my = lax.axis_index("i")
        left = lax.rem(my + N_DEV - 1, N_DEV)
        right = lax.rem(my + 1, N_DEV)

        barrier_sem = pltpu.get_barrier_semaphore()
        for nbr in (left, right):
            pl.semaphore_signal(
                barrier_sem, inc=1,
                device_id=(nbr,), device_id_type=pl.DeviceIdType.MESH,
            )
        pl.semaphore_wait(barrier_sem, 2)

        Q = jnp.dot(x_ref[...], wq_ref[...],
                    preferred_element_type=jnp.float32)
        K = k_ref[...]
        V = v_ref[...]
        Wo_l = wo_ref[...]

        qi = lax.broadcasted_iota(jnp.int32, (Sq, Skv), 0)
        ki = lax.broadcasted_iota(jnp.int32, (Sq, Skv), 1)
        mask = (jnp.abs(qi - ki) <= 128) | (ki < 32) | (qi < 32)

        for b in range(B):
            acc = jnp.zeros((Sq, D_MODEL), jnp.float32)
            for h in range(HQ_LOCAL):
                q_bh = Q[b * Sq:(b + 1) * Sq, h * DH:(h + 1) * DH]
                k_bh = K[b, :, h * DH:(h + 1) * DH]
                s = lax.dot_general(
                    q_bh, k_bh, (((1,), (1,)), ((), ())),
                    preferred_element_type=jnp.float32) * 0.125
                s = jnp.where(mask, s, jnp.float32(-1e9))
                m = jnp.max(s, axis=1, keepdims=True)
                w = jnp.exp(s - m)
                w = w / jnp.sum(w, axis=1, keepdims=True)
                ctx = jnp.dot(w, V[b, :, h * DH:(h + 1) * DH],
                              preferred_element_type=jnp.float32)
                acc = acc + jnp.dot(ctx, Wo_l[h * DH:(h + 1) * DH, :],
                                    preferred_element_type=jnp.float32)
            out_ref[b, :, :] = acc
            comm_ref[0, b, :, :] = acc

        for h in range(N_DEV - 1):
            rdma = pltpu.make_async_remote_copy(
                src_ref=comm_ref.at[h],
                dst_ref=comm_ref.at[h + 1],
                send_sem=send_sems.at[h],
                recv_sem=recv_sems.at[h],
                device_id=(right,),
                device_id_type=pl.DeviceIdType.MESH,
            )
            rdma.start()
            rdma.wait()
            out_ref[...] = out_ref[...] + comm_ref[h + 1]

    return pl.pallas_call(
        body,
        out_shape=jax.ShapeDtypeStruct((B, Sq, D_MODEL), jnp.float32),
        in_specs=[pl.BlockSpec(memory_space=pltpu.VMEM)] * 5,
        out_specs=pl.BlockSpec(memory_space=pltpu.VMEM),
        scratch_shapes=[
            pltpu.VMEM((N_DEV, B, Sq, D_MODEL), jnp.float32),
            pltpu.SemaphoreType.DMA((N_DEV - 1,)),
            pltpu.SemaphoreType.DMA((N_DEV - 1,)),
        ],
        compiler_params=pltpu.CompilerParams(collective_id=0),
    )(x2, Wq, K2, V2, Wo)


# device time: 25140 ns/iter; 1.9709x vs baseline; 1.9709x over previous
import jax
import jax.numpy as jnp
from jax import lax
from jax.experimental import pallas as pl
from jax.experimental.pallas import tpu as pltpu

N_DEV = 4
B, Sq, Skv = 2, 256, 256
HQ_LOCAL, DH = 4, 64
D_MODEL = 512
RB = 32
NBLK = Sq // RB


def kernel(x, Wq, K_ext, V_ext, Wo):
    my_pos = lax.axis_index("i")
    K_loc = lax.dynamic_slice_in_dim(K_ext, my_pos * HQ_LOCAL, HQ_LOCAL, axis=2)
    V_loc = lax.dynamic_slice_in_dim(V_ext, my_pos * HQ_LOCAL, HQ_LOCAL, axis=2)
    K2 = K_loc.reshape(B, Skv, HQ_LOCAL * DH)
    V2 = V_loc.reshape(B, Skv, HQ_LOCAL * DH)
    x2 = x.reshape(B * Sq, D_MODEL)

    def body(x_ref, wq_ref, k_ref, v_ref, wo_ref, out_ref,
             work_ref, r1_ref, r2_ref, send_sems, recv_sems):
        my = lax.axis_index("i")
        pa = my ^ 1
        pb = 3 - my

        o1 = (my ^ (my >> 1)) & 1
        o2 = (my >> 1) & 1
        p = my & 1

        s1_own_h = o1 * 2
        s1_snd_h = (1 - o1) * 2
        s1_own_q = o1 * 2 + p
        s1_snd_q = o1 * 2 + (1 - p)
        s2_own_h = 4 + o2 * 2
        s2_snd_h = 4 + (1 - o2) * 2
        s2_own_q = 4 + o2 * 2 + p
        s2_snd_q = 4 + o2 * 2 + (1 - p)

        barrier_sem = pltpu.get_barrier_semaphore()
        for nbr in (pa, pb):
            pl.semaphore_signal(
                barrier_sem, inc=1,
                device_id=(nbr,), device_id_type=pl.DeviceIdType.MESH,
            )
        pl.semaphore_wait(barrier_sem, 2)

        Q = jnp.dot(x_ref[...], wq_ref[...],
                    preferred_element_type=jnp.float32)
        K = k_ref[...]
        V = v_ref[...]
        Wo_l = wo_ref[...]

        qi = lax.broadcasted_iota(jnp.int32, (Sq, Skv), 0)
        ki = lax.broadcasted_iota(jnp.int32, (Sq, Skv), 1)
        mask = (jnp.abs(qi - ki) <= 128) | (ki < 32) | (qi < 32)

        for b in range(B):
            acc = jnp.zeros((Sq, D_MODEL), jnp.float32)
            for h in range(HQ_LOCAL):
                q_bh = Q[b * Sq:(b + 1) * Sq, h * DH:(h + 1) * DH]
                k_bh = K[b, :, h * DH:(h + 1) * DH]
                s = lax.dot_general(
                    q_bh, k_bh, (((1,), (1,)), ((), ())),
                    preferred_element_type=jnp.float32) * 0.125
                s = jnp.where(mask, s, jnp.float32(-1e9))
                m = jnp.max(s, axis=1, keepdims=True)
                w = jnp.exp(s - m)
                w = w / jnp.sum(w, axis=1, keepdims=True)
                ctx = jnp.dot(w, V[b, :, h * DH:(h + 1) * DH],
                              preferred_element_type=jnp.float32)
                acc = acc + jnp.dot(ctx, Wo_l[h * DH:(h + 1) * DH, :],
                                    preferred_element_type=jnp.float32)
            for rb in range(NBLK):
                work_ref[rb, b, :, :] = acc[rb * RB:(rb + 1) * RB, :]

        def xchg(phase, stream, src_blk, nblk, dst_ref_slice, partner):
            return pltpu.make_async_remote_copy(
                src_ref=work_ref.at[pl.ds(src_blk, nblk)],
                dst_ref=dst_ref_slice,
                send_sem=send_sems.at[phase, stream],
                recv_sem=recv_sems.at[phase, stream],
                device_id=(partner,),
                device_id_type=pl.DeviceIdType.MESH,
            )

        d11 = xchg(0, 0, s1_snd_h, 2, r1_ref.at[0], pa)
        d12 = xchg(0, 1, s2_snd_h, 2, r1_ref.at[1], pb)
        d11.start()
        d12.start()
        d11.wait()
        d12.wait()
        work_ref[pl.ds(s1_own_h, 2)] = work_ref[pl.ds(s1_own_h, 2)] + r1_ref[0]
        work_ref[pl.ds(s2_own_h, 2)] = work_ref[pl.ds(s2_own_h, 2)] + r1_ref[1]

        d21 = xchg(1, 0, s1_snd_q, 1, r2_ref.at[0], pb)
        d22 = xchg(1, 1, s2_snd_q, 1, r2_ref.at[1], pa)
        d21.start()
        d22.start()
        d21.wait()
        d22.wait()
        work_ref[pl.ds(s1_own_q, 1)] = work_ref[pl.ds(s1_own_q, 1)] + r2_ref[0]
        work_ref[pl.ds(s2_own_q, 1)] = work_ref[pl.ds(s2_own_q, 1)] + r2_ref[1]

        d31 = xchg(2, 0, s1_own_q, 1, work_ref.at[pl.ds(s1_own_q, 1)], pb)
        d32 = xchg(2, 1, s2_own_q, 1, work_ref.at[pl.ds(s2_own_q, 1)], pa)
        d31.start()
        d32.start()
        d31.wait()
        d32.wait()

        d41 = xchg(3, 0, s1_own_h, 2, work_ref.at[pl.ds(s1_own_h, 2)], pa)
        d42 = xchg(3, 1, s2_own_h, 2, work_ref.at[pl.ds(s2_own_h, 2)], pb)
        d41.start()
        d42.start()
        d41.wait()
        d42.wait()

        for rb in range(NBLK):
            for b in range(B):
                out_ref[b, rb * RB:(rb + 1) * RB, :] = work_ref[rb, b, :, :]

    return pl.pallas_call(
        body,
        out_shape=jax.ShapeDtypeStruct((B, Sq, D_MODEL), jnp.float32),
        in_specs=[pl.BlockSpec(memory_space=pltpu.VMEM)] * 5,
        out_specs=pl.BlockSpec(memory_space=pltpu.VMEM),
        scratch_shapes=[
            pltpu.VMEM((NBLK, B, RB, D_MODEL), jnp.float32),
            pltpu.VMEM((2, 2, B, RB, D_MODEL), jnp.float32),
            pltpu.VMEM((2, 1, B, RB, D_MODEL), jnp.float32),
            pltpu.SemaphoreType.DMA((4, 2)),
            pltpu.SemaphoreType.DMA((4, 2)),
        ],
        compiler_params=pltpu.CompilerParams(collective_id=0),
    )(x2, Wq, K2, V2, Wo)


# device time: 24666 ns/iter; 2.0088x vs baseline; 1.0192x over previous
import jax
import jax.numpy as jnp
from jax import lax
from jax.experimental import pallas as pl
from jax.experimental.pallas import tpu as pltpu

N_DEV = 4
B, Sq, Skv = 2, 256, 256
HQ_LOCAL, DH = 4, 64
D_MODEL = 512


def kernel(x, Wq, K_ext, V_ext, Wo):
    my_pos = lax.axis_index("i")
    K_loc = lax.dynamic_slice_in_dim(K_ext, my_pos * HQ_LOCAL, HQ_LOCAL, axis=2)
    V_loc = lax.dynamic_slice_in_dim(V_ext, my_pos * HQ_LOCAL, HQ_LOCAL, axis=2)
    K2 = K_loc.reshape(B, Skv, HQ_LOCAL * DH)
    V2 = V_loc.reshape(B, Skv, HQ_LOCAL * DH)
    x2 = x.reshape(B * Sq, D_MODEL)

    def body(x_ref, wq_ref, k_ref, v_ref, wo_ref, out_ref,
             p1_ref, p2_ref, send_sems, recv_sems):
        my = lax.axis_index("i")
        pa = my ^ 1
        pb = 3 - my

        barrier_sem = pltpu.get_barrier_semaphore()
        for nbr in (pa, pb):
            pl.semaphore_signal(
                barrier_sem, inc=1,
                device_id=(nbr,), device_id_type=pl.DeviceIdType.MESH,
            )
        pl.semaphore_wait(barrier_sem, 2)

        K = k_ref[...]
        V = v_ref[...]
        Wo_l = wo_ref[...]

        qi = lax.broadcasted_iota(jnp.int32, (Sq, Skv), 0)
        ki = lax.broadcasted_iota(jnp.int32, (Sq, Skv), 1)
        mask = (jnp.abs(qi - ki) <= 128) | (ki < 32) | (qi < 32)

        def partial_for_batch(b):
            Qb = jnp.dot(x_ref[b * Sq:(b + 1) * Sq, :], wq_ref[...],
                         preferred_element_type=jnp.float32)
            acc = jnp.zeros((Sq, D_MODEL), jnp.float32)
            for h in range(HQ_LOCAL):
                q_bh = Qb[:, h * DH:(h + 1) * DH]
                k_bh = K[b, :, h * DH:(h + 1) * DH]
                s = lax.dot_general(
                    q_bh, k_bh, (((1,), (1,)), ((), ())),
                    preferred_element_type=jnp.float32) * 0.125
                s = jnp.where(mask, s, jnp.float32(-1e9))
                m = jnp.max(s, axis=1, keepdims=True)
                w = jnp.exp(s - m)
                w = w / jnp.sum(w, axis=1, keepdims=True)
                ctx = jnp.dot(w, V[b, :, h * DH:(h + 1) * DH],
                              preferred_element_type=jnp.float32)
                acc = acc + jnp.dot(ctx, Wo_l[h * DH:(h + 1) * DH, :],
                                    preferred_element_type=jnp.float32)
            return acc

        def xchg(phase, half, dst_ref_slice, partner):
            return pltpu.make_async_remote_copy(
                src_ref=out_ref.at[half],
                dst_ref=dst_ref_slice,
                send_sem=send_sems.at[phase, half],
                recv_sem=recv_sems.at[phase, half],
                device_id=(partner,),
                device_id_type=pl.DeviceIdType.MESH,
            )

        out_ref[0, :, :] = partial_for_batch(0)
        d1l = xchg(0, 0, p1_ref.at[0], pa)
        d1l.start()

        out_ref[1, :, :] = partial_for_batch(1)
        d1r = xchg(0, 1, p1_ref.at[1], pb)
        d1r.start()

        d1l.wait()
        out_ref[0, :, :] = out_ref[0, :, :] + p1_ref[0]
        d2l = xchg(1, 0, p2_ref.at[0], pb)
        d2l.start()

        d1r.wait()
        out_ref[1, :, :] = out_ref[1, :, :] + p1_ref[1]
        d2r = xchg(1, 1, p2_ref.at[1], pa)
        d2r.start()

        d2l.wait()
        out_ref[0, :, :] = out_ref[0, :, :] + p2_ref[0]
        d2r.wait()
        out_ref[1, :, :] = out_ref[1, :, :] + p2_ref[1]

    return pl.pallas_call(
        body,
        out_shape=jax.ShapeDtypeStruct((B, Sq, D_MODEL), jnp.float32),
        in_specs=[pl.BlockSpec(memory_space=pltpu.VMEM)] * 5,
        out_specs=pl.BlockSpec(memory_space=pltpu.VMEM),
        scratch_shapes=[
            pltpu.VMEM((B, Sq, D_MODEL), jnp.float32),
            pltpu.VMEM((B, Sq, D_MODEL), jnp.float32),
            pltpu.SemaphoreType.DMA((2, 2)),
            pltpu.SemaphoreType.DMA((2, 2)),
        ],
        compiler_params=pltpu.CompilerParams(collective_id=0),
    )(x2, Wq, K2, V2, Wo)
